# baseline (speedup 1.0000x reference)
import functools

import jax
import jax.numpy as jnp
from jax import lax
from jax.experimental import pallas as pl
from jax.experimental.pallas import tpu as pltpu
from jax.experimental.pallas import tpu_sc as plsc

N_GRAPH, N_NODE, N_FEAT = 1024, 128, 9
HIDDEN = 64
NUM_WORKERS = 32
GRAPHS_PER_WORKER = N_GRAPH // NUM_WORKERS
LANES = 16
VPR = HIDDEN // LANES
HALF = N_NODE // 2
ROWS_PER_HALF = HALF * N_FEAT
IDX_ROWS = ROWS_PER_HALF // HALF


def _sc_kernel():
    mesh = plsc.VectorSubcoreMesh(core_axis_name="c", subcore_axis_name="s")

    @functools.partial(
        pl.kernel,
        mesh=mesh,
        out_type=jax.ShapeDtypeStruct((N_GRAPH, N_NODE + 1, HIDDEN), jnp.float32),
        scratch_types=[
            pltpu.VMEM((2, IDX_ROWS, HALF), jnp.int32),
            pltpu.VMEM((2, 2, HALF), jnp.int32),
            pltpu.VMEM((2, ROWS_PER_HALF, HIDDEN), jnp.float32),
            pltpu.VMEM((2, 2, HALF, HIDDEN), jnp.float32),
            pltpu.VMEM((N_NODE + 1, HIDDEN), jnp.float32),
            pltpu.SemaphoreType.DMA,
            pltpu.SemaphoreType.DMA,
            pltpu.SemaphoreType.DMA,
        ],
        compiler_params=pltpu.CompilerParams(use_tc_tiling_on_sc=False),
    )
    def k(x_r, in_deg, out_deg, atom_t, in_t, out_t, token,
          out, idxa, idxd, rows, drows, obuf, sem0, sem1, semw):
        wid = lax.axis_index("s") * 2 + lax.axis_index("c")
        sems = (sem0, sem1)

        def load_idx(g, s, h):
            pltpu.sync_copy(x_r.at[g, h], idxa.at[s])
            pltpu.sync_copy(in_deg.at[g, h], idxd.at[s, 0])
            pltpu.sync_copy(out_deg.at[g, h], idxd.at[s, 1])

        def fire(s):
            sem = sems[s]
            for a in range(IDX_ROWS):
                pltpu.async_copy(
                    atom_t.at[idxa.at[s, a]],
                    rows.at[s, pl.ds(a * HALF, HALF)], sem)
            pltpu.async_copy(in_t.at[idxd.at[s, 0]], drows.at[s, 0], sem)
            pltpu.async_copy(out_t.at[idxd.at[s, 1]], drows.at[s, 1], sem)

        def drain(s):
            sem = sems[s]
            for a in range(IDX_ROWS):
                pltpu.make_async_copy(
                    atom_t.at[idxa.at[s, a]],
                    rows.at[s, pl.ds(a * HALF, HALF)], sem).wait()
            pltpu.make_async_copy(in_t.at[idxd.at[s, 0]], drows.at[s, 0], sem).wait()
            pltpu.make_async_copy(out_t.at[idxd.at[s, 1]], drows.at[s, 1], sem).wait()

        def sum_half(s, h):
            base = h * HALF + 1

            def per_node(n, nc):
                m = n * N_FEAT
                for v in range(VPR):
                    sl = pl.ds(v * LANES, LANES)
                    acc = rows[s, m, sl]
                    for j in range(1, N_FEAT):
                        acc = acc + rows[s, m + j, sl]
                    acc = acc + drows[s, 0, n, sl] + drows[s, 1, n, sl]
                    obuf[base + n, sl] = acc
                return nc

            lax.fori_loop(0, HALF, per_node, 0)

        pltpu.sync_copy(token, obuf.at[pl.ds(0, 1)])
        g0 = wid * GRAPHS_PER_WORKER
        load_idx(g0, 0, 0)
        fire(0)

        def per_graph(t, carry):
            g = g0 + t
            load_idx(g, 1, 1)
            fire(1)
            drain(0)

            @pl.when(t > 0)
            def _():
                pltpu.make_async_copy(obuf, out.at[g - 1], semw).wait()

            sum_half(0, 0)

            @pl.when(t < GRAPHS_PER_WORKER - 1)
            def _():
                load_idx(g + 1, 0, 0)
                fire(0)

            drain(1)
            sum_half(1, 1)
            pltpu.async_copy(obuf, out.at[g], semw)
            return carry

        lax.fori_loop(0, GRAPHS_PER_WORKER, per_graph, 0)
        pltpu.make_async_copy(
            obuf, out.at[g0 + GRAPHS_PER_WORKER - 1], semw).wait()

    return k


def kernel(x, in_degree, out_degree, atom_table, in_table, out_table, graph_token_w):
    x_r = x.astype(jnp.int32).reshape(N_GRAPH, 2, IDX_ROWS, HALF)
    in_d = in_degree.astype(jnp.int32).reshape(N_GRAPH, 2, HALF)
    out_d = out_degree.astype(jnp.int32).reshape(N_GRAPH, 2, HALF)
    return _sc_kernel()(
        x_r,
        in_d,
        out_d,
        atom_table,
        in_table,
        out_table,
        graph_token_w,
    )


# speedup vs baseline: 1.3065x; 1.3065x over previous
import functools

import jax
import jax.numpy as jnp
from jax import lax
from jax.experimental import pallas as pl
from jax.experimental.pallas import tpu as pltpu
from jax.experimental.pallas import tpu_sc as plsc

N_GRAPH, N_NODE, N_FEAT = 1024, 128, 9
HIDDEN = 64
NUM_WORKERS = 32
GRAPHS_PER_WORKER = N_GRAPH // NUM_WORKERS
LANES = 16
VPR = HIDDEN // LANES
HALF = N_NODE // 2
ROWS_PER_HALF = HALF * N_FEAT
IDX_ROWS = ROWS_PER_HALF // HALF


def _sc_kernel():
    mesh = plsc.VectorSubcoreMesh(core_axis_name="c", subcore_axis_name="s")

    @functools.partial(
        pl.kernel,
        mesh=mesh,
        out_type=jax.ShapeDtypeStruct((N_GRAPH, N_NODE + 1, HIDDEN), jnp.float32),
        scratch_types=[
            pltpu.VMEM((2, IDX_ROWS, HALF), jnp.int32),
            pltpu.VMEM((2, 2, HALF), jnp.int32),
            pltpu.VMEM((2, ROWS_PER_HALF, HIDDEN), jnp.float32),
            pltpu.VMEM((2, 2, HALF, HIDDEN), jnp.float32),
            pltpu.VMEM((N_NODE + 1, HIDDEN), jnp.float32),
            pltpu.SemaphoreType.DMA,
            pltpu.SemaphoreType.DMA,
            pltpu.SemaphoreType.DMA,
        ],
        compiler_params=pltpu.CompilerParams(use_tc_tiling_on_sc=False),
    )
    def k(x_r, in_deg, out_deg, atom_t, in_t, out_t, token,
          out, idxa, idxd, rows, drows, obuf, sem0, sem1, semw):
        wid = lax.axis_index("s") * 2 + lax.axis_index("c")
        sems = (sem0, sem1)

        def load_idx(g, s, h):
            pltpu.sync_copy(x_r.at[g, h], idxa.at[s])
            pltpu.sync_copy(in_deg.at[g, h], idxd.at[s, 0])
            pltpu.sync_copy(out_deg.at[g, h], idxd.at[s, 1])

        def fire(s):
            sem = sems[s]
            for a in range(IDX_ROWS):
                pltpu.async_copy(
                    atom_t.at[idxa.at[s, a]],
                    rows.at[s, pl.ds(a * HALF, HALF)], sem)
            pltpu.async_copy(in_t.at[idxd.at[s, 0]], drows.at[s, 0], sem)
            pltpu.async_copy(out_t.at[idxd.at[s, 1]], drows.at[s, 1], sem)

        def drain(s):
            sem = sems[s]
            for a in range(IDX_ROWS):
                pltpu.make_async_copy(
                    atom_t.at[idxa.at[s, a]],
                    rows.at[s, pl.ds(a * HALF, HALF)], sem).wait()
            pltpu.make_async_copy(in_t.at[idxd.at[s, 0]], drows.at[s, 0], sem).wait()
            pltpu.make_async_copy(out_t.at[idxd.at[s, 1]], drows.at[s, 1], sem).wait()

        def sum_half(s, h):
            base = h * HALF + 1

            def per_node(n, nc):
                m = n * N_FEAT
                cols = []
                for v in range(VPR):
                    sl = pl.ds(v * LANES, LANES)
                    vals = [rows[s, m + j, sl] for j in range(N_FEAT)]
                    vals.append(drows[s, 0, n, sl])
                    vals.append(drows[s, 1, n, sl])
                    cols.append(vals)
                outs = []
                for vals in cols:
                    while len(vals) > 1:
                        nxt = [vals[i] + vals[i + 1]
                               for i in range(0, len(vals) - 1, 2)]
                        if len(vals) % 2:
                            nxt.append(vals[-1])
                        vals = nxt
                    outs.append(vals[0])
                for v in range(VPR):
                    obuf[base + n, pl.ds(v * LANES, LANES)] = outs[v]
                return nc

            lax.fori_loop(0, HALF, per_node, 0)

        pltpu.sync_copy(token, obuf.at[pl.ds(0, 1)])
        g0 = wid * GRAPHS_PER_WORKER
        load_idx(g0, 0, 0)
        fire(0)

        def per_graph(t, carry):
            g = g0 + t
            load_idx(g, 1, 1)
            fire(1)
            drain(0)

            @pl.when(t > 0)
            def _():
                pltpu.make_async_copy(obuf, out.at[g - 1], semw).wait()

            sum_half(0, 0)

            @pl.when(t < GRAPHS_PER_WORKER - 1)
            def _():
                load_idx(g + 1, 0, 0)
                fire(0)

            drain(1)
            sum_half(1, 1)
            pltpu.async_copy(obuf, out.at[g], semw)
            return carry

        lax.fori_loop(0, GRAPHS_PER_WORKER, per_graph, 0)
        pltpu.make_async_copy(
            obuf, out.at[g0 + GRAPHS_PER_WORKER - 1], semw).wait()

    return k


def kernel(x, in_degree, out_degree, atom_table, in_table, out_table, graph_token_w):
    x_r = x.astype(jnp.int32).reshape(N_GRAPH, 2, IDX_ROWS, HALF)
    in_d = in_degree.astype(jnp.int32).reshape(N_GRAPH, 2, HALF)
    out_d = out_degree.astype(jnp.int32).reshape(N_GRAPH, 2, HALF)
    return _sc_kernel()(
        x_r,
        in_d,
        out_d,
        atom_table,
        in_table,
        out_table,
        graph_token_w,
    )


# speedup vs baseline: 1.5065x; 1.1531x over previous
import functools

import jax
import jax.numpy as jnp
from jax import lax
from jax.experimental import pallas as pl
from jax.experimental.pallas import tpu as pltpu
from jax.experimental.pallas import tpu_sc as plsc

N_GRAPH, N_NODE, N_FEAT = 1024, 128, 9
HIDDEN = 64
NUM_WORKERS = 32
GRAPHS_PER_WORKER = N_GRAPH // NUM_WORKERS


def _sc_kernel():
    mesh = plsc.VectorSubcoreMesh(core_axis_name="c", subcore_axis_name="s")

    @functools.partial(
        pl.kernel,
        mesh=mesh,
        out_type=jax.ShapeDtypeStruct((N_GRAPH, N_NODE + 1, HIDDEN), jnp.float32),
        scratch_types=[
            pltpu.VMEM((2, N_FEAT, N_NODE), jnp.int32),
            pltpu.VMEM((2, 2, N_NODE), jnp.int32),
            pltpu.VMEM((2, N_NODE, HIDDEN), jnp.float32),
            pltpu.VMEM((1, HIDDEN), jnp.float32),
            pltpu.SemaphoreType.DMA,
            pltpu.SemaphoreType.DMA,
            pltpu.SemaphoreType.DMA,
            pltpu.SemaphoreType.DMA,
            pltpu.SemaphoreType.DMA,
            pltpu.SemaphoreType.DMA,
            pltpu.SemaphoreType.DMA,
        ],
        compiler_params=pltpu.CompilerParams(use_tc_tiling_on_sc=False),
    )
    def k(x_p, in_deg, out_deg, atom_t, in_t, out_t, token, zeros,
          out, idxa, idxd, acc, tokbuf,
          semg0, semg1, semo, semi0, semi1, semz0, semz1):
        wid = lax.axis_index("s") * 2 + lax.axis_index("c")
        semg = (semg0, semg1)
        semi = (semi0, semi1)
        semz = (semz0, semz1)
        g0 = wid * GRAPHS_PER_WORKER

        def prep_start(g, s):
            pltpu.async_copy(zeros, acc.at[s], semz[s])
            for j in range(N_FEAT):
                pltpu.async_copy(x_p.at[j, g], idxa.at[s, j], semi[s])
            pltpu.async_copy(in_deg.at[g], idxd.at[s, 0], semi[s])
            pltpu.async_copy(out_deg.at[g], idxd.at[s, 1], semi[s])

        def prep_finish(g, s):
            pltpu.make_async_copy(zeros, acc.at[s], semz[s]).wait()
            for j in range(N_FEAT):
                pltpu.make_async_copy(x_p.at[j, g], idxa.at[s, j], semi[s]).wait()
            pltpu.make_async_copy(in_deg.at[g], idxd.at[s, 0], semi[s]).wait()
            pltpu.make_async_copy(out_deg.at[g], idxd.at[s, 1], semi[s]).wait()

        def fire_adds(s):
            for j in range(N_FEAT):
                pltpu.async_copy(atom_t.at[idxa.at[s, j]], acc.at[s],
                                 semg[s], add=True)
            pltpu.async_copy(in_t.at[idxd.at[s, 0]], acc.at[s], semg[s], add=True)
            pltpu.async_copy(out_t.at[idxd.at[s, 1]], acc.at[s], semg[s], add=True)

        def drain_adds(s):
            for j in range(N_FEAT):
                pltpu.make_async_copy(atom_t.at[idxa.at[s, j]], acc.at[s],
                                      semg[s]).wait()
            pltpu.make_async_copy(in_t.at[idxd.at[s, 0]], acc.at[s], semg[s]).wait()
            pltpu.make_async_copy(out_t.at[idxd.at[s, 1]], acc.at[s], semg[s]).wait()

        def fire_out(g, s):
            pltpu.async_copy(acc.at[s], out.at[g, pl.ds(1, N_NODE)], semo)
            pltpu.async_copy(tokbuf, out.at[g, pl.ds(0, 1)], semo)

        def drain_out(g, s):
            pltpu.make_async_copy(acc.at[s], out.at[g, pl.ds(1, N_NODE)],
                                  semo).wait()
            pltpu.make_async_copy(tokbuf, out.at[g, pl.ds(0, 1)], semo).wait()

        pltpu.sync_copy(token, tokbuf)
        for s in (0, 1):
            prep_start(g0 + s, s)
        for s in (0, 1):
            prep_finish(g0 + s, s)
            fire_adds(s)

        def per_pair(t, carry):
            ga = g0 + 2 * t
            for s in (0, 1):
                g = ga + s
                drain_adds(s)
                fire_out(g, s)
                drain_out(g, s)

                @pl.when(t < GRAPHS_PER_WORKER // 2 - 1)
                def _():
                    prep_start(g + 2, s)
                    prep_finish(g + 2, s)
                    fire_adds(s)
            return carry

        lax.fori_loop(0, GRAPHS_PER_WORKER // 2, per_pair, 0)

    return k


def kernel(x, in_degree, out_degree, atom_table, in_table, out_table, graph_token_w):
    x_p = jnp.transpose(x.astype(jnp.int32), (2, 0, 1))
    zeros = jnp.zeros((N_NODE, HIDDEN), jnp.float32)
    return _sc_kernel()(
        x_p,
        in_degree.astype(jnp.int32),
        out_degree.astype(jnp.int32),
        atom_table,
        in_table,
        out_table,
        graph_token_w,
        zeros,
    )
